# baseline (speedup 1.0000x reference)
import functools

import jax
import jax.numpy as jnp
from jax import lax
from jax.experimental import pallas as pl
from jax.experimental.pallas import tpu as pltpu
from jax.experimental.pallas import tpu_sc as plsc

N = 10000
D = 128
OUT = 64
E = 160000
E_TRAIN = 100000
E_REG = 150000
E_REG_PAD = 153600
E_TR_PAD = 102400
K_EDGE = 8000
N_CHUNK = E // K_EDGE
N_GROUP = K_EDGE // 16
NTILES = 32

_F32 = jnp.float32


def _mm_body(x_ref, wt_ref, wb_ref, b_ref, lab_ref, plab_ref, nlab_ref,
             yt_ref, bt_ref, wp_ref, wn_ref):
    x = x_ref[...]
    yt_ref[...] = jax.lax.dot_general(
        wt_ref[...], x, (((1,), (1,)), ((), ())), preferred_element_type=_F32)
    bt_ref[...] = jax.lax.dot_general(
        wb_ref[...], x, (((1,), (1,)), ((), ())),
        preferred_element_type=_F32) + b_ref[...]
    l0 = lab_ref[0:1, 0:1]
    l1 = lab_ref[0:1, 1:2]
    l2 = lab_ref[0:1, 2:3]
    pl_ = plab_ref[...]
    nl_ = nlab_ref[...]
    wp_ref[...] = jnp.where(pl_ == 0, l0, jnp.where(pl_ == 1, l1, l2))
    wn_ref[...] = jnp.where(nl_ == 0, l0, jnp.where(nl_ == 1, l1, l2))


def _run_mm(x, wtT, wbT, b2, lab2, plab2, nlab2):
    return pl.pallas_call(
        _mm_body,
        out_shape=[
            jax.ShapeDtypeStruct((OUT, N), _F32),
            jax.ShapeDtypeStruct((OUT, N), _F32),
            jax.ShapeDtypeStruct((1, E), _F32),
            jax.ShapeDtypeStruct((1, E), _F32),
        ],
    )(x, wtT, wbT, b2, lab2, plab2, nlab2)


def _agg_body(yt_h, pe_h, ne_h,
              stp_h, stn_h, cp_h, cn_h,
              y0, y1, s0p, s1p, s0n, s1n, cp, cn, ebufa, ebufb,
              sema, semb, semy):
    wid = lax.axis_index("s") * 2 + lax.axis_index("c")
    f0 = 2 * wid

    pltpu.sync_copy(yt_h.at[pl.ds(f0 * N, N)], y0)
    pltpu.sync_copy(yt_h.at[pl.ds((f0 + 1) * N, N)], y1)

    zeros16 = jnp.zeros((16,), _F32)

    def _zero(i, _):
        sl = pl.ds(i * 16, 16)
        s0p[sl] = zeros16
        s1p[sl] = zeros16
        s0n[sl] = zeros16
        s1n[sl] = zeros16
        cp[sl] = zeros16
        cn[sl] = zeros16
        return 0

    lax.fori_loop(0, N // 16, _zero, 0)

    ones16 = jnp.full((16,), 1.0, _F32)

    KB = 2 * K_EDGE

    def _process(e_h, s0, s1, cnt, do_cnt):
        def _consume(buf):
            def _group(g, _):
                v = buf[pl.ds(g * 16, 16)]
                sv = jnp.bitwise_and(v, 0xFFFF)
                dv = lax.shift_right_logical(v, 16)
                w = plsc.bitcast(buf[pl.ds(K_EDGE + g * 16, 16)], _F32)
                plsc.addupdate_scatter(s0, [dv], plsc.load_gather(y0, [sv]) * w)
                plsc.addupdate_scatter(s1, [dv], plsc.load_gather(y1, [sv]) * w)
                if do_cnt:
                    plsc.addupdate_scatter(cnt, [dv], ones16)
                return 0

            lax.fori_loop(0, N_GROUP, _group, 0)

        pltpu.async_copy(e_h.at[pl.ds(0, KB)], ebufa, sema)

        def _pair(pi, _):
            pltpu.async_copy(e_h.at[pl.ds((2 * pi + 1) * KB, KB)], ebufb, semb)
            pltpu.make_async_copy(e_h.at[pl.ds(0, KB)], ebufa, sema).wait()
            _consume(ebufa)
            pltpu.async_copy(e_h.at[pl.ds((2 * pi + 2) * KB, KB)], ebufa, sema)
            pltpu.make_async_copy(e_h.at[pl.ds(0, KB)], ebufb, semb).wait()
            _consume(ebufb)
            return 0

        lax.fori_loop(0, N_CHUNK // 2, _pair, 0)
        pltpu.make_async_copy(e_h.at[pl.ds(0, KB)], ebufa, sema).wait()

    cidx = lax.axis_index("c")

    @pl.when(cidx == 0)
    def _():
        _process(pe_h, s0p, s1p, cp, True)
        _process(ne_h, s0n, s1n, cn, False)

    @pl.when(cidx == 1)
    def _():
        _process(pe_h, s0p, s1p, cp, False)
        _process(ne_h, s0n, s1n, cn, True)

    pltpu.sync_copy(s0p, stp_h.at[pl.ds(f0 * N, N)])
    pltpu.sync_copy(s1p, stp_h.at[pl.ds((f0 + 1) * N, N)])
    pltpu.sync_copy(s0n, stn_h.at[pl.ds(f0 * N, N)])
    pltpu.sync_copy(s1n, stn_h.at[pl.ds((f0 + 1) * N, N)])

    @pl.when(wid == 0)
    def _():
        pltpu.sync_copy(cp, cp_h)

    @pl.when(wid == 1)
    def _():
        pltpu.sync_copy(cn, cn_h)


def _run_agg(yt_flat, pe, ne):
    mesh = plsc.VectorSubcoreMesh(core_axis_name="c", subcore_axis_name="s", num_cores=2, num_subcores=16)
    f = pl.kernel(
        _agg_body,
        out_type=[
            jax.ShapeDtypeStruct((OUT * N,), _F32),
            jax.ShapeDtypeStruct((OUT * N,), _F32),
            jax.ShapeDtypeStruct((N,), _F32),
            jax.ShapeDtypeStruct((N,), _F32),
        ],
        mesh=mesh,
        compiler_params=pltpu.CompilerParams(needs_layout_passes=False),
        scratch_types=[
            pltpu.VMEM((N,), _F32),
            pltpu.VMEM((N,), _F32),
            pltpu.VMEM((N,), _F32),
            pltpu.VMEM((N,), _F32),
            pltpu.VMEM((N,), _F32),
            pltpu.VMEM((N,), _F32),
            pltpu.VMEM((N,), _F32),
            pltpu.VMEM((N,), _F32),
            pltpu.VMEM((2 * K_EDGE,), jnp.int32),
            pltpu.VMEM((2 * K_EDGE,), jnp.int32),
            pltpu.SemaphoreType.DMA,
            pltpu.SemaphoreType.DMA,
            pltpu.SemaphoreType.DMA,
        ],
    )
    return f(yt_flat, pe, ne)


def _post_body(sp_ref, sn_ref, bt_ref, cpr, cnr, rw_ref, wtop_ref, wbot_ref,
               rwo_ref, zt_ref, pt_ref, qt_ref, st_ref):
    bt = bt_ref[...]
    hp = jnp.tanh(sp_ref[...] / jnp.maximum(cpr[...], 1.0) + bt)
    hn = jnp.tanh(sn_ref[...] / jnp.maximum(cnr[...], 1.0) + bt)
    hcat = jnp.concatenate([hp, hn], axis=0)
    z = jnp.tanh(jax.lax.dot_general(
        hcat, rw_ref[...], (((0,), (0,)), ((), ())),
        preferred_element_type=_F32))
    zt_ref[...] = z
    pt_ref[...] = jax.lax.dot_general(
        wtop_ref[...], z, (((1,), (1,)), ((), ())), preferred_element_type=_F32)
    qt_ref[...] = jax.lax.dot_general(
        wbot_ref[...], z, (((1,), (1,)), ((), ())), preferred_element_type=_F32)
    st_ref[...] = jax.nn.sigmoid(jax.lax.dot_general(
        rwo_ref[...], z, (((1,), (1,)), ((), ())), preferred_element_type=_F32))


def _run_post(stp, stn, bt, cp2, cn2, rwT, wtopT, wbotT, rwoT):
    return pl.pallas_call(
        _post_body,
        out_shape=[
            jax.ShapeDtypeStruct((N, 2 * OUT), _F32),
            jax.ShapeDtypeStruct((3, N), _F32),
            jax.ShapeDtypeStruct((3, N), _F32),
            jax.ShapeDtypeStruct((1, N), _F32),
        ],
    )(stp, stn, bt, cp2, cn2, rwT, wtopT, wbotT, rwoT)


def _loss_body(pt_h, qt_h, st_h, se0_h, se1_h, te0_h, te1_h, ytr_h,
               pred_h, emb_h,
               p0, p1, p2, q0, q1, q2, st,
               se0b, se1b, pb0, pb1, pb2, te0b, te1b, ytb, accv):
    wid = lax.axis_index("s") * 2 + lax.axis_index("c")

    pltpu.sync_copy(pt_h.at[pl.ds(0 * N, N)], p0)
    pltpu.sync_copy(pt_h.at[pl.ds(1 * N, N)], p1)
    pltpu.sync_copy(pt_h.at[pl.ds(2 * N, N)], p2)
    pltpu.sync_copy(qt_h.at[pl.ds(0 * N, N)], q0)
    pltpu.sync_copy(qt_h.at[pl.ds(1 * N, N)], q1)
    pltpu.sync_copy(qt_h.at[pl.ds(2 * N, N)], q2)
    pltpu.sync_copy(st_h, st)

    nreg = E_REG_PAD // NTILES
    base = wid * nreg
    pltpu.sync_copy(se0_h.at[pl.ds(base, nreg)], se0b)
    pltpu.sync_copy(se1_h.at[pl.ds(base, nreg)], se1b)

    def _reg(g, _):
        sl = pl.ds(g * 16, 16)
        e0 = se0b[sl]
        e1 = se1b[sl]
        pb0[sl] = plsc.load_gather(p0, [e0]) + plsc.load_gather(q0, [e1])
        pb1[sl] = plsc.load_gather(p1, [e0]) + plsc.load_gather(q1, [e1])
        pb2[sl] = plsc.load_gather(p2, [e0]) + plsc.load_gather(q2, [e1])
        return 0

    lax.fori_loop(0, nreg // 16, _reg, 0)
    pltpu.sync_copy(pb0, pred_h.at[pl.ds(0 * E_REG_PAD + base, nreg)])
    pltpu.sync_copy(pb1, pred_h.at[pl.ds(1 * E_REG_PAD + base, nreg)])
    pltpu.sync_copy(pb2, pred_h.at[pl.ds(2 * E_REG_PAD + base, nreg)])

    ntr = E_TR_PAD // NTILES
    base2 = wid * ntr
    pltpu.sync_copy(te0_h.at[pl.ds(base2, ntr)], te0b)
    pltpu.sync_copy(te1_h.at[pl.ds(base2, ntr)], te1b)
    pltpu.sync_copy(ytr_h.at[pl.ds(base2, ntr)], ytb)

    def _emb(g, acc):
        sl = pl.ds(g * 16, 16)
        d = plsc.load_gather(st, [te0b[sl]]) - plsc.load_gather(st, [te1b[sl]])
        pos = ytb[sl] > 0
        qv = jnp.where(pos, jnp.minimum(d, -0.5), jnp.maximum(d, 0.5))
        t = qv - d
        valid = (base2 + g * 16) < E_TRAIN
        return acc + jnp.where(valid, t * t, jnp.zeros((16,), _F32))

    acc = lax.fori_loop(0, ntr // 16, _emb, jnp.zeros((16,), _F32))
    accv[...] = acc
    pltpu.sync_copy(accv, emb_h.at[pl.ds(wid * 16, 16)])


def _run_loss(pt_flat, qt_flat, status, se0, se1, te0, te1, ytr):
    mesh = plsc.VectorSubcoreMesh(core_axis_name="c", subcore_axis_name="s", num_cores=2, num_subcores=16)
    nreg = E_REG_PAD // NTILES
    ntr = E_TR_PAD // NTILES
    f = pl.kernel(
        _loss_body,
        out_type=[
            jax.ShapeDtypeStruct((3 * E_REG_PAD,), _F32),
            jax.ShapeDtypeStruct((NTILES * 16,), _F32),
        ],
        mesh=mesh,
        compiler_params=pltpu.CompilerParams(needs_layout_passes=False),
        scratch_types=[
            pltpu.VMEM((N,), _F32),
            pltpu.VMEM((N,), _F32),
            pltpu.VMEM((N,), _F32),
            pltpu.VMEM((N,), _F32),
            pltpu.VMEM((N,), _F32),
            pltpu.VMEM((N,), _F32),
            pltpu.VMEM((N,), _F32),
            pltpu.VMEM((nreg,), jnp.int32),
            pltpu.VMEM((nreg,), jnp.int32),
            pltpu.VMEM((nreg,), _F32),
            pltpu.VMEM((nreg,), _F32),
            pltpu.VMEM((nreg,), _F32),
            pltpu.VMEM((ntr,), jnp.int32),
            pltpu.VMEM((ntr,), jnp.int32),
            pltpu.VMEM((ntr,), jnp.int32),
            pltpu.VMEM((16,), _F32),
        ],
    )
    return f(pt_flat, qt_flat, status, se0, se1, te0, te1, ytr)


def _red_body(pred_ref, tgt_ref, emb_ref, out_ref, acc):
    i = pl.program_id(0)
    p = pred_ref[...]
    t = tgt_ref[...]
    nb = p.shape[1]
    m = jnp.max(p, axis=0, keepdims=True)
    lse = jnp.log(jnp.sum(jnp.exp(p - m), axis=0, keepdims=True)) + m
    rows = lax.broadcasted_iota(jnp.int32, (3, nb), 0)
    ptgt = jnp.sum(jnp.where(rows == t, p, 0.0), axis=0, keepdims=True)
    cols = lax.broadcasted_iota(jnp.int32, (1, nb), 1) + i * nb
    bs = jnp.sum(jnp.where(cols < E_REG, lse - ptgt, 0.0))

    @pl.when(i == 0)
    def _():
        acc[0] = jnp.sum(emb_ref[...]) / float(E_TRAIN)

    acc[0] = acc[0] + bs / float(E_REG)

    @pl.when(i == pl.num_programs(0) - 1)
    def _():
        out_ref[...] = jnp.full((1, 1), acc[0], _F32)


def _run_red(predt, tgt2, emb2):
    nb = 2048
    grid = (E_REG_PAD // nb,)
    return pl.pallas_call(
        _red_body,
        grid=grid,
        in_specs=[
            pl.BlockSpec((3, nb), lambda i: (0, i)),
            pl.BlockSpec((1, nb), lambda i: (0, i)),
            pl.BlockSpec((1, NTILES * 16), lambda i: (0, 0)),
        ],
        out_specs=pl.BlockSpec((1, 1), lambda i: (0, 0)),
        out_shape=jax.ShapeDtypeStruct((1, 1), _F32),
        scratch_shapes=[pltpu.SMEM((1,), _F32)],
    )(predt, tgt2, emb2)


def kernel(X, positive_edges, negative_edges, train_edges, y_train,
           positive_labels, negative_labels, sorted_train_edges, target,
           W_in, b_in, lab_in, W_out, b_out, lab_out,
           regression_weights, regression_weight, regression_weights_out):
    wtT = W_in[:D].T
    wbT = W_in[D:].T
    b2 = b_in.reshape(OUT, 1)
    lab2 = lab_in.reshape(1, 3)
    wtopT = regression_weights[:2 * OUT].T
    wbotT = regression_weights[2 * OUT:].T
    rwoT = regression_weights_out.T

    se0 = sorted_train_edges[:, :, 0].reshape(E_REG)
    se1 = sorted_train_edges[:, :, 1].reshape(E_REG)
    se0 = jnp.pad(se0, (0, E_REG_PAD - E_REG))
    se1 = jnp.pad(se1, (0, E_REG_PAD - E_REG))
    te0 = jnp.pad(train_edges[0], (0, E_TR_PAD - E_TRAIN))
    te1 = jnp.pad(train_edges[1], (0, E_TR_PAD - E_TRAIN))
    ytr = jnp.pad(y_train, (0, E_TR_PAD - E_TRAIN), constant_values=1)
    tgt2 = jnp.pad(target, (0, E_REG_PAD - E_REG)).reshape(1, E_REG_PAD)

    yt, bt, wp, wn = _run_mm(
        X, wtT, wbT, b2, lab2,
        positive_labels.reshape(1, E), negative_labels.reshape(1, E))

    def _pack(edges, w):
        wb = jax.lax.bitcast_convert_type(w.reshape(E), jnp.int32)
        sd = jnp.bitwise_or(jnp.left_shift(edges[1], 16), edges[0])
        e = jnp.stack([sd, wb])
        e = e.reshape(2, N_CHUNK, K_EDGE).transpose(1, 0, 2).reshape(-1)
        return jnp.pad(e, (0, 2 * 2 * K_EDGE))

    pe = _pack(positive_edges, wp)
    ne = _pack(negative_edges, wn)

    stp, stn, cp, cn = _run_agg(yt.reshape(OUT * N), pe, ne)

    z, pt, qt, st = _run_post(
        stp.reshape(OUT, N), stn.reshape(OUT, N), bt,
        cp.reshape(1, N), cn.reshape(1, N), regression_weight,
        wtopT, wbotT, rwoT)

    predt, emb = _run_loss(
        pt.reshape(3 * N), qt.reshape(3 * N), st.reshape(N),
        se0, se1, te0, te1, ytr)

    loss = _run_red(predt.reshape(3, E_REG_PAD), tgt2,
                    emb.reshape(1, NTILES * 16))[0, 0]

    return (loss, z)


# speedup vs baseline: 1.0799x; 1.0799x over previous
import functools

import jax
import jax.numpy as jnp
from jax import lax
from jax.experimental import pallas as pl
from jax.experimental.pallas import tpu as pltpu
from jax.experimental.pallas import tpu_sc as plsc

N = 10000
D = 128
OUT = 64
E = 160000
E_TRAIN = 100000
E_REG = 150000
E_REG_PAD = 153600
E_TR_PAD = 102400
K_EDGE = 4000
N_CHUNK = E // K_EDGE
N_GROUP = K_EDGE // 16
NTILES = 32

_F32 = jnp.float32


def _mm_body(x_ref, wt_ref, wb_ref, b_ref, lab_ref, plab_ref, nlab_ref,
             yt_ref, bt_ref, wp_ref, wn_ref):
    x = x_ref[...]
    yt_ref[...] = jax.lax.dot_general(
        wt_ref[...], x, (((1,), (1,)), ((), ())), preferred_element_type=_F32)
    bt_ref[...] = jax.lax.dot_general(
        wb_ref[...], x, (((1,), (1,)), ((), ())),
        preferred_element_type=_F32) + b_ref[...]
    l0 = lab_ref[0:1, 0:1]
    l1 = lab_ref[0:1, 1:2]
    l2 = lab_ref[0:1, 2:3]
    pl_ = plab_ref[...]
    nl_ = nlab_ref[...]
    wp_ref[...] = jnp.where(pl_ == 0, l0, jnp.where(pl_ == 1, l1, l2))
    wn_ref[...] = jnp.where(nl_ == 0, l0, jnp.where(nl_ == 1, l1, l2))


def _run_mm(x, wtT, wbT, b2, lab2, plab2, nlab2):
    return pl.pallas_call(
        _mm_body,
        out_shape=[
            jax.ShapeDtypeStruct((OUT, N), _F32),
            jax.ShapeDtypeStruct((OUT, N), _F32),
            jax.ShapeDtypeStruct((1, E), _F32),
            jax.ShapeDtypeStruct((1, E), _F32),
        ],
    )(x, wtT, wbT, b2, lab2, plab2, nlab2)


def _agg_body(yt_h, pe_h, ne_h,
              stp_h, stn_h, cp_h, cn_h,
              y0, y1, s0p, s1p, s0n, s1n, cp, cn, ebufa, ebufb,
              sema, semb, semy):
    wid = lax.axis_index("s") * 2 + lax.axis_index("c")
    f0 = 2 * wid

    pltpu.sync_copy(yt_h.at[pl.ds(f0 * N, N)], y0)
    pltpu.sync_copy(yt_h.at[pl.ds((f0 + 1) * N, N)], y1)

    zeros16 = jnp.zeros((16,), _F32)

    def _zero(i, _):
        sl = pl.ds(i * 16, 16)
        s0p[sl] = zeros16
        s1p[sl] = zeros16
        s0n[sl] = zeros16
        s1n[sl] = zeros16
        cp[sl] = zeros16
        cn[sl] = zeros16
        return 0

    lax.fori_loop(0, N // 16, _zero, 0)

    ones16 = jnp.full((16,), 1.0, _F32)

    KB = 3 * K_EDGE

    def _process(e_h, s0, s1, cnt, do_cnt):
        def _consume(buf):
            def _group(g, _):
                sv = buf[pl.ds(g * 16, 16)]
                dv = buf[pl.ds(K_EDGE + g * 16, 16)]
                w = plsc.bitcast(buf[pl.ds(2 * K_EDGE + g * 16, 16)], _F32)
                plsc.addupdate_scatter(s0, [dv], plsc.load_gather(y0, [sv]) * w)
                plsc.addupdate_scatter(s1, [dv], plsc.load_gather(y1, [sv]) * w)
                if do_cnt:
                    plsc.addupdate_scatter(cnt, [dv], ones16)
                return 0

            lax.fori_loop(0, N_GROUP, _group, 0)

        pltpu.async_copy(e_h.at[pl.ds(0, KB)], ebufa, sema)

        def _pair(pi, _):
            pltpu.async_copy(e_h.at[pl.ds((2 * pi + 1) * KB, KB)], ebufb, semb)
            pltpu.make_async_copy(e_h.at[pl.ds(0, KB)], ebufa, sema).wait()
            _consume(ebufa)
            pltpu.async_copy(e_h.at[pl.ds((2 * pi + 2) * KB, KB)], ebufa, sema)
            pltpu.make_async_copy(e_h.at[pl.ds(0, KB)], ebufb, semb).wait()
            _consume(ebufb)
            return 0

        lax.fori_loop(0, N_CHUNK // 2, _pair, 0)
        pltpu.make_async_copy(e_h.at[pl.ds(0, KB)], ebufa, sema).wait()

    cidx = lax.axis_index("c")

    @pl.when(cidx == 0)
    def _():
        _process(pe_h, s0p, s1p, cp, True)
        _process(ne_h, s0n, s1n, cn, False)

    @pl.when(cidx == 1)
    def _():
        _process(pe_h, s0p, s1p, cp, False)
        _process(ne_h, s0n, s1n, cn, True)

    pltpu.sync_copy(s0p, stp_h.at[pl.ds(f0 * N, N)])
    pltpu.sync_copy(s1p, stp_h.at[pl.ds((f0 + 1) * N, N)])
    pltpu.sync_copy(s0n, stn_h.at[pl.ds(f0 * N, N)])
    pltpu.sync_copy(s1n, stn_h.at[pl.ds((f0 + 1) * N, N)])

    @pl.when(wid == 0)
    def _():
        pltpu.sync_copy(cp, cp_h)

    @pl.when(wid == 1)
    def _():
        pltpu.sync_copy(cn, cn_h)


def _run_agg(yt_flat, pe, ne):
    mesh = plsc.VectorSubcoreMesh(core_axis_name="c", subcore_axis_name="s", num_cores=2, num_subcores=16)
    f = pl.kernel(
        _agg_body,
        out_type=[
            jax.ShapeDtypeStruct((OUT * N,), _F32),
            jax.ShapeDtypeStruct((OUT * N,), _F32),
            jax.ShapeDtypeStruct((N,), _F32),
            jax.ShapeDtypeStruct((N,), _F32),
        ],
        mesh=mesh,
        compiler_params=pltpu.CompilerParams(needs_layout_passes=False),
        scratch_types=[
            pltpu.VMEM((N,), _F32),
            pltpu.VMEM((N,), _F32),
            pltpu.VMEM((N,), _F32),
            pltpu.VMEM((N,), _F32),
            pltpu.VMEM((N,), _F32),
            pltpu.VMEM((N,), _F32),
            pltpu.VMEM((N,), _F32),
            pltpu.VMEM((N,), _F32),
            pltpu.VMEM((3 * K_EDGE,), jnp.int32),
            pltpu.VMEM((3 * K_EDGE,), jnp.int32),
            pltpu.SemaphoreType.DMA,
            pltpu.SemaphoreType.DMA,
            pltpu.SemaphoreType.DMA,
        ],
    )
    return f(yt_flat, pe, ne)


def _post_body(sp_ref, sn_ref, bt_ref, cpr, cnr, rw_ref, wtop_ref, wbot_ref,
               rwo_ref, zt_ref, pt_ref, qt_ref, st_ref):
    bt = bt_ref[...]
    hp = jnp.tanh(sp_ref[...] / jnp.maximum(cpr[...], 1.0) + bt)
    hn = jnp.tanh(sn_ref[...] / jnp.maximum(cnr[...], 1.0) + bt)
    hcat = jnp.concatenate([hp, hn], axis=0)
    z = jnp.tanh(jax.lax.dot_general(
        hcat, rw_ref[...], (((0,), (0,)), ((), ())),
        preferred_element_type=_F32))
    zt_ref[...] = z
    pt_ref[...] = jax.lax.dot_general(
        wtop_ref[...], z, (((1,), (1,)), ((), ())), preferred_element_type=_F32)
    qt_ref[...] = jax.lax.dot_general(
        wbot_ref[...], z, (((1,), (1,)), ((), ())), preferred_element_type=_F32)
    st_ref[...] = jax.nn.sigmoid(jax.lax.dot_general(
        rwo_ref[...], z, (((1,), (1,)), ((), ())), preferred_element_type=_F32))


def _run_post(stp, stn, bt, cp2, cn2, rwT, wtopT, wbotT, rwoT):
    return pl.pallas_call(
        _post_body,
        out_shape=[
            jax.ShapeDtypeStruct((N, 2 * OUT), _F32),
            jax.ShapeDtypeStruct((3, N), _F32),
            jax.ShapeDtypeStruct((3, N), _F32),
            jax.ShapeDtypeStruct((1, N), _F32),
        ],
    )(stp, stn, bt, cp2, cn2, rwT, wtopT, wbotT, rwoT)


def _loss_body(pt_h, qt_h, st_h, se0_h, se1_h, te0_h, te1_h, ytr_h,
               pred_h, emb_h,
               p0, p1, p2, q0, q1, q2, st,
               se0b, se1b, pb0, pb1, pb2, te0b, te1b, ytb, accv):
    wid = lax.axis_index("s") * 2 + lax.axis_index("c")

    pltpu.sync_copy(pt_h.at[pl.ds(0 * N, N)], p0)
    pltpu.sync_copy(pt_h.at[pl.ds(1 * N, N)], p1)
    pltpu.sync_copy(pt_h.at[pl.ds(2 * N, N)], p2)
    pltpu.sync_copy(qt_h.at[pl.ds(0 * N, N)], q0)
    pltpu.sync_copy(qt_h.at[pl.ds(1 * N, N)], q1)
    pltpu.sync_copy(qt_h.at[pl.ds(2 * N, N)], q2)
    pltpu.sync_copy(st_h, st)

    nreg = E_REG_PAD // NTILES
    base = wid * nreg
    pltpu.sync_copy(se0_h.at[pl.ds(base, nreg)], se0b)
    pltpu.sync_copy(se1_h.at[pl.ds(base, nreg)], se1b)

    def _reg(g, _):
        sl = pl.ds(g * 16, 16)
        e0 = se0b[sl]
        e1 = se1b[sl]
        pb0[sl] = plsc.load_gather(p0, [e0]) + plsc.load_gather(q0, [e1])
        pb1[sl] = plsc.load_gather(p1, [e0]) + plsc.load_gather(q1, [e1])
        pb2[sl] = plsc.load_gather(p2, [e0]) + plsc.load_gather(q2, [e1])
        return 0

    lax.fori_loop(0, nreg // 16, _reg, 0)
    pltpu.sync_copy(pb0, pred_h.at[pl.ds(0 * E_REG_PAD + base, nreg)])
    pltpu.sync_copy(pb1, pred_h.at[pl.ds(1 * E_REG_PAD + base, nreg)])
    pltpu.sync_copy(pb2, pred_h.at[pl.ds(2 * E_REG_PAD + base, nreg)])

    ntr = E_TR_PAD // NTILES
    base2 = wid * ntr
    pltpu.sync_copy(te0_h.at[pl.ds(base2, ntr)], te0b)
    pltpu.sync_copy(te1_h.at[pl.ds(base2, ntr)], te1b)
    pltpu.sync_copy(ytr_h.at[pl.ds(base2, ntr)], ytb)

    def _emb(g, acc):
        sl = pl.ds(g * 16, 16)
        d = plsc.load_gather(st, [te0b[sl]]) - plsc.load_gather(st, [te1b[sl]])
        pos = ytb[sl] > 0
        qv = jnp.where(pos, jnp.minimum(d, -0.5), jnp.maximum(d, 0.5))
        t = qv - d
        valid = (base2 + g * 16) < E_TRAIN
        return acc + jnp.where(valid, t * t, jnp.zeros((16,), _F32))

    acc = lax.fori_loop(0, ntr // 16, _emb, jnp.zeros((16,), _F32))
    accv[...] = acc
    pltpu.sync_copy(accv, emb_h.at[pl.ds(wid * 16, 16)])


def _run_loss(pt_flat, qt_flat, status, se0, se1, te0, te1, ytr):
    mesh = plsc.VectorSubcoreMesh(core_axis_name="c", subcore_axis_name="s", num_cores=2, num_subcores=16)
    nreg = E_REG_PAD // NTILES
    ntr = E_TR_PAD // NTILES
    f = pl.kernel(
        _loss_body,
        out_type=[
            jax.ShapeDtypeStruct((3 * E_REG_PAD,), _F32),
            jax.ShapeDtypeStruct((NTILES * 16,), _F32),
        ],
        mesh=mesh,
        compiler_params=pltpu.CompilerParams(needs_layout_passes=False),
        scratch_types=[
            pltpu.VMEM((N,), _F32),
            pltpu.VMEM((N,), _F32),
            pltpu.VMEM((N,), _F32),
            pltpu.VMEM((N,), _F32),
            pltpu.VMEM((N,), _F32),
            pltpu.VMEM((N,), _F32),
            pltpu.VMEM((N,), _F32),
            pltpu.VMEM((nreg,), jnp.int32),
            pltpu.VMEM((nreg,), jnp.int32),
            pltpu.VMEM((nreg,), _F32),
            pltpu.VMEM((nreg,), _F32),
            pltpu.VMEM((nreg,), _F32),
            pltpu.VMEM((ntr,), jnp.int32),
            pltpu.VMEM((ntr,), jnp.int32),
            pltpu.VMEM((ntr,), jnp.int32),
            pltpu.VMEM((16,), _F32),
        ],
    )
    return f(pt_flat, qt_flat, status, se0, se1, te0, te1, ytr)


def _red_body(pred_ref, tgt_ref, emb_ref, out_ref, acc):
    i = pl.program_id(0)
    p = pred_ref[...]
    t = tgt_ref[...]
    nb = p.shape[1]
    m = jnp.max(p, axis=0, keepdims=True)
    lse = jnp.log(jnp.sum(jnp.exp(p - m), axis=0, keepdims=True)) + m
    rows = lax.broadcasted_iota(jnp.int32, (3, nb), 0)
    ptgt = jnp.sum(jnp.where(rows == t, p, 0.0), axis=0, keepdims=True)
    cols = lax.broadcasted_iota(jnp.int32, (1, nb), 1) + i * nb
    bs = jnp.sum(jnp.where(cols < E_REG, lse - ptgt, 0.0))

    @pl.when(i == 0)
    def _():
        acc[0] = jnp.sum(emb_ref[...]) / float(E_TRAIN)

    acc[0] = acc[0] + bs / float(E_REG)

    @pl.when(i == pl.num_programs(0) - 1)
    def _():
        out_ref[...] = jnp.full((1, 1), acc[0], _F32)


def _run_red(predt, tgt2, emb2):
    nb = 2048
    grid = (E_REG_PAD // nb,)
    return pl.pallas_call(
        _red_body,
        grid=grid,
        in_specs=[
            pl.BlockSpec((3, nb), lambda i: (0, i)),
            pl.BlockSpec((1, nb), lambda i: (0, i)),
            pl.BlockSpec((1, NTILES * 16), lambda i: (0, 0)),
        ],
        out_specs=pl.BlockSpec((1, 1), lambda i: (0, 0)),
        out_shape=jax.ShapeDtypeStruct((1, 1), _F32),
        scratch_shapes=[pltpu.SMEM((1,), _F32)],
    )(predt, tgt2, emb2)


def kernel(X, positive_edges, negative_edges, train_edges, y_train,
           positive_labels, negative_labels, sorted_train_edges, target,
           W_in, b_in, lab_in, W_out, b_out, lab_out,
           regression_weights, regression_weight, regression_weights_out):
    wtT = W_in[:D].T
    wbT = W_in[D:].T
    b2 = b_in.reshape(OUT, 1)
    lab2 = lab_in.reshape(1, 3)
    wtopT = regression_weights[:2 * OUT].T
    wbotT = regression_weights[2 * OUT:].T
    rwoT = regression_weights_out.T

    se0 = sorted_train_edges[:, :, 0].reshape(E_REG)
    se1 = sorted_train_edges[:, :, 1].reshape(E_REG)
    se0 = jnp.pad(se0, (0, E_REG_PAD - E_REG))
    se1 = jnp.pad(se1, (0, E_REG_PAD - E_REG))
    te0 = jnp.pad(train_edges[0], (0, E_TR_PAD - E_TRAIN))
    te1 = jnp.pad(train_edges[1], (0, E_TR_PAD - E_TRAIN))
    ytr = jnp.pad(y_train, (0, E_TR_PAD - E_TRAIN), constant_values=1)
    tgt2 = jnp.pad(target, (0, E_REG_PAD - E_REG)).reshape(1, E_REG_PAD)

    yt, bt, wp, wn = _run_mm(
        X, wtT, wbT, b2, lab2,
        positive_labels.reshape(1, E), negative_labels.reshape(1, E))

    def _pack(edges, w):
        wb = jax.lax.bitcast_convert_type(w.reshape(E), jnp.int32)
        e = jnp.concatenate([edges, wb[None, :]], axis=0)
        e = e.reshape(3, N_CHUNK, K_EDGE).transpose(1, 0, 2).reshape(-1)
        return jnp.pad(e, (0, 2 * 3 * K_EDGE))

    pe = _pack(positive_edges, wp)
    ne = _pack(negative_edges, wn)

    stp, stn, cp, cn = _run_agg(yt.reshape(OUT * N), pe, ne)

    z, pt, qt, st = _run_post(
        stp.reshape(OUT, N), stn.reshape(OUT, N), bt,
        cp.reshape(1, N), cn.reshape(1, N), regression_weight,
        wtopT, wbotT, rwoT)

    predt, emb = _run_loss(
        pt.reshape(3 * N), qt.reshape(3 * N), st.reshape(N),
        se0, se1, te0, te1, ytr)

    loss = _run_red(predt.reshape(3, E_REG_PAD), tgt2,
                    emb.reshape(1, NTILES * 16))[0, 0]

    return (loss, z)
